# baseline (speedup 1.0000x reference)
import functools

import jax
import jax.numpy as jnp
from jax import lax
from jax.experimental import pallas as pl
from jax.experimental.pallas import tpu as pltpu
from jax.experimental.pallas import tpu_sc as plsc

B = 16384
ED = 32
HL = 64
G = 20
NC, NS = 2, 16
NW = NC * NS
BPW = B // NW
CHUNK = 128
NCH = BPW // CHUNK

@functools.cache
def _sc_gather_fn():
    mesh = plsc.VectorSubcoreMesh(core_axis_name="c", subcore_axis_name="s",
                                  num_cores=NC, num_subcores=NS)

    @functools.partial(
        pl.kernel,
        out_type=(
            jax.ShapeDtypeStruct((B, ED), jnp.float32),
            jax.ShapeDtypeStruct((B, ED), jnp.float32),
            jax.ShapeDtypeStruct((B,), jnp.float32),
            jax.ShapeDtypeStruct((B,), jnp.float32),
        ),
        mesh=mesh,
        compiler_params=pltpu.CompilerParams(use_tc_tiling_on_sc=False),
        scratch_types=(
            pltpu.VMEM((NCH, CHUNK), jnp.int32),
            pltpu.VMEM((NCH, CHUNK), jnp.int32),
            pltpu.VMEM((NCH, CHUNK), jnp.int32),
            pltpu.VMEM((NCH, CHUNK), jnp.int32),
            pltpu.VMEM((2, CHUNK, 4 * ED), jnp.float32),
            pltpu.VMEM((2, CHUNK, 4 * ED), jnp.float32),
            pltpu.VMEM((BPW, ED), jnp.float32),
            pltpu.VMEM((BPW, ED), jnp.float32),
            pltpu.VMEM((BPW,), jnp.float32),
            pltpu.VMEM((BPW,), jnp.float32),
            pltpu.SemaphoreType.DMA,
            pltpu.SemaphoreType.DMA,
            pltpu.SemaphoreType.DMA,
            pltpu.SemaphoreType.DMA,
            pltpu.SemaphoreType.DMA,
        ),
    )
    def _sc_gather(uidx_hbm, midx_hbm, uemb_hbm, memb_hbm, ubias_hbm, mbias_hbm,
                   urows_out, mrows_out, ub_out, mb_out,
                   uidx_v, midx_v, uphys_v, mphys_v, ubuf, mbuf, uout_v, mout_v,
                   ub_v, mb_v, sem_gu0, sem_gu1, sem_gm0, sem_gm1, sem_b):
        wid = lax.axis_index("s") * NC + lax.axis_index("c")
        base = wid * BPW
        sem_gu = (sem_gu0, sem_gu1)
        sem_gm = (sem_gm0, sem_gm1)
        pltpu.sync_copy(uidx_hbm.at[wid], uidx_v)
        pltpu.sync_copy(midx_hbm.at[wid], midx_v)
        bias_copies = []
        for j in range(NCH):
            sl = pl.ds(j * CHUNK, CHUNK)
            bias_copies.append(pltpu.async_copy(ubias_hbm.at[uidx_v.at[j]], ub_v.at[sl], sem_b))
            bias_copies.append(pltpu.async_copy(mbias_hbm.at[midx_v.at[j]], mb_v.at[sl], sem_b))
        for j in range(NCH):
            for k in range(CHUNK // 16):
                sl = pl.ds(k * 16, 16)
                uphys_v[j, sl] = lax.shift_right_logical(uidx_v[j, sl], 2)
                mphys_v[j, sl] = lax.shift_right_logical(midx_v[j, sl], 2)

        def extract(buf, idxref, outref, j):
            def body(g, carry):
                offs = (idxref[j, pl.ds(g * 16, 16)] & 3) * ED
                for k in range(16):
                    off = offs[k]
                    r = g * 16 + k
                    row = j * CHUNK + r
                    outref[row, pl.ds(0, 16)] = buf[r, pl.ds(off, 16)]
                    outref[row, pl.ds(16, 16)] = buf[r, pl.ds(off + 16, 16)]
                return carry
            lax.fori_loop(0, CHUNK // 16, body, 0)

        gu = [None] * NCH
        gm = [None] * NCH
        gu[0] = pltpu.async_copy(uemb_hbm.at[uphys_v.at[0]], ubuf.at[0], sem_gu[0])
        gm[0] = pltpu.async_copy(memb_hbm.at[mphys_v.at[0]], mbuf.at[0], sem_gm[0])
        for j in range(NCH):
            if j + 1 < NCH:
                p = (j + 1) % 2
                gu[j + 1] = pltpu.async_copy(uemb_hbm.at[uphys_v.at[j + 1]], ubuf.at[p], sem_gu[p])
                gm[j + 1] = pltpu.async_copy(memb_hbm.at[mphys_v.at[j + 1]], mbuf.at[p], sem_gm[p])
            gu[j].wait()
            extract(ubuf.at[j % 2], uidx_v, uout_v, j)
            gm[j].wait()
            extract(mbuf.at[j % 2], midx_v, mout_v, j)

        for c in bias_copies:
            c.wait()
        pltpu.sync_copy(uout_v, urows_out.at[pl.ds(base, BPW)])
        pltpu.sync_copy(mout_v, mrows_out.at[pl.ds(base, BPW)])
        pltpu.sync_copy(ub_v, ub_out.at[pl.ds(base, BPW)])
        pltpu.sync_copy(mb_v, mb_out.at[pl.ds(base, BPW)])

    return _sc_gather


def _tc_body(u_ref, m_ref, g_ref, ub_ref, mb_ref, gW_ref, gb_ref,
             w1u_ref, w1m_ref, w1g_ref, b1_ref, w2_ref, c2_ref, out_ref):
    cdims = (((1,), (1,)), ((), ()))
    u = u_ref[...]
    m = m_ref[...]
    ge = lax.dot_general(g_ref[...], gW_ref[...], cdims,
                         preferred_element_type=jnp.float32) + gb_ref[...]
    acc = lax.dot_general(u, w1u_ref[...], cdims, preferred_element_type=jnp.float32)
    acc = acc + lax.dot_general(m, w1m_ref[...], cdims, preferred_element_type=jnp.float32)
    acc = acc + lax.dot_general(ge, w1g_ref[...], cdims, preferred_element_type=jnp.float32)
    h = jnp.maximum(acc + b1_ref[...], 0.0)
    mlp = lax.dot_general(h, w2_ref[...], cdims, preferred_element_type=jnp.float32)
    dp = jnp.sum(u * m, axis=1)
    out_ref[...] = dp + mlp[:, 0] + ub_ref[...] + mb_ref[...] + c2_ref[0, 0]


BLK = 2048


def _tc_forward(urows, mrows, genre, ub, mb, gW, gb2, w1u, w1m, w1g, b12, W2, c2):
    return pl.pallas_call(
        _tc_body,
        grid=(B // BLK,),
        in_specs=[
            pl.BlockSpec((BLK, ED), lambda i: (i, 0)),
            pl.BlockSpec((BLK, ED), lambda i: (i, 0)),
            pl.BlockSpec((BLK, G), lambda i: (i, 0)),
            pl.BlockSpec((BLK,), lambda i: (i,)),
            pl.BlockSpec((BLK,), lambda i: (i,)),
            pl.BlockSpec((ED, G), lambda i: (0, 0)),
            pl.BlockSpec((1, ED), lambda i: (0, 0)),
            pl.BlockSpec((HL, ED), lambda i: (0, 0)),
            pl.BlockSpec((HL, ED), lambda i: (0, 0)),
            pl.BlockSpec((HL, ED), lambda i: (0, 0)),
            pl.BlockSpec((1, HL), lambda i: (0, 0)),
            pl.BlockSpec((1, HL), lambda i: (0, 0)),
            pl.BlockSpec((1, 1), lambda i: (0, 0)),
        ],
        out_specs=pl.BlockSpec((BLK,), lambda i: (i,)),
        out_shape=jax.ShapeDtypeStruct((B,), jnp.float32),
    )(urows, mrows, genre, ub, mb, gW, gb2, w1u, w1m, w1g, b12, W2, c2)


def kernel(userIndices, movieIndices, genreIndeces, userEmb, movieEmb,
           userBiasT, movieBiasT, bias, gW, gb, W1, b1, W2, b2):
    uidx = userIndices.astype(jnp.int32).reshape(NW, NCH, CHUNK)
    midx = movieIndices.astype(jnp.int32).reshape(NW, NCH, CHUNK)
    urows, mrows, ub, mb = _sc_gather_fn()(
        uidx, midx, userEmb.reshape(-1, 4 * ED), movieEmb.reshape(-1, 4 * ED),
        userBiasT[:, 0], movieBiasT[:, 0])
    w1u = W1[:, :ED]
    w1m = W1[:, ED:2 * ED]
    w1g = W1[:, 2 * ED:]
    c2 = (bias + b2).reshape(1, 1)
    return _tc_forward(urows, mrows, genreIndeces, ub, mb, gW,
                       gb.reshape(1, ED), w1u, w1m, w1g,
                       b1.reshape(1, HL), W2, c2)


# speedup vs baseline: 3.8013x; 3.8013x over previous
import functools

import jax
import jax.numpy as jnp
from jax import lax
from jax.experimental import pallas as pl
from jax.experimental.pallas import tpu as pltpu
from jax.experimental.pallas import tpu_sc as plsc

B = 16384
ED = 32
HL = 64
G = 20
NC, NS = 2, 16
NW = NC * NS
BPW = B // NW
CHUNK = 128
NCH = BPW // CHUNK


@functools.cache
def _sc_gather_fn():
    mesh = plsc.VectorSubcoreMesh(core_axis_name="c", subcore_axis_name="s",
                                  num_cores=NC, num_subcores=NS)

    @functools.partial(
        pl.kernel,
        out_type=(
            jax.ShapeDtypeStruct((B, ED), jnp.float32),
            jax.ShapeDtypeStruct((B,), jnp.float32),
            jax.ShapeDtypeStruct((B,), jnp.float32),
        ),
        mesh=mesh,
        compiler_params=pltpu.CompilerParams(use_tc_tiling_on_sc=False),
        scratch_types=(
            pltpu.VMEM((NCH, CHUNK), jnp.int32),
            pltpu.VMEM((NCH, CHUNK), jnp.int32),
            pltpu.VMEM((BPW, ED), jnp.float32),
            pltpu.VMEM((BPW,), jnp.float32),
            pltpu.VMEM((BPW,), jnp.float32),
            pltpu.SemaphoreType.DMA,
        ),
    )
    def _sc_gather(uidx_hbm, midx_hbm, memb_hbm, ubias_hbm, mbias_hbm,
                   mrows_out, ub_out, mb_out,
                   uidx_v, midx_v, mrows_v, ub_v, mb_v, sem):
        wid = lax.axis_index("s") * NC + lax.axis_index("c")
        base = wid * BPW
        pltpu.sync_copy(uidx_hbm.at[wid], uidx_v)
        pltpu.sync_copy(midx_hbm.at[wid], midx_v)
        copies = []
        for j in range(NCH):
            sl = pl.ds(j * CHUNK, CHUNK)
            copies.append(pltpu.async_copy(memb_hbm.at[midx_v.at[j]], mrows_v.at[sl], sem))
            copies.append(pltpu.async_copy(ubias_hbm.at[uidx_v.at[j]], ub_v.at[sl], sem))
            copies.append(pltpu.async_copy(mbias_hbm.at[midx_v.at[j]], mb_v.at[sl], sem))
        for c in copies:
            c.wait()
        pltpu.sync_copy(mrows_v, mrows_out.at[pl.ds(base, BPW)])
        pltpu.sync_copy(ub_v, ub_out.at[pl.ds(base, BPW)])
        pltpu.sync_copy(mb_v, mb_out.at[pl.ds(base, BPW)])

    return _sc_gather


def _tc_body(u_ref, m_ref, g_ref, ub_ref, mb_ref, gW_ref, gb_ref,
             w1u_ref, w1m_ref, w1g_ref, b1_ref, w2_ref, c2_ref, out_ref):
    cdims = (((1,), (1,)), ((), ()))
    u = u_ref[...]
    m = m_ref[...]
    ge = lax.dot_general(g_ref[...], gW_ref[...], cdims,
                         preferred_element_type=jnp.float32) + gb_ref[...]
    acc = lax.dot_general(u, w1u_ref[...], cdims, preferred_element_type=jnp.float32)
    acc = acc + lax.dot_general(m, w1m_ref[...], cdims, preferred_element_type=jnp.float32)
    acc = acc + lax.dot_general(ge, w1g_ref[...], cdims, preferred_element_type=jnp.float32)
    h = jnp.maximum(acc + b1_ref[...], 0.0)
    mlp = lax.dot_general(h, w2_ref[...], cdims, preferred_element_type=jnp.float32)
    dp = jnp.sum(u * m, axis=1)
    out_ref[...] = dp + mlp[:, 0] + ub_ref[...] + mb_ref[...] + c2_ref[0, 0]


BLK = 2048


def _tc_forward(urows, mrows, genre, ub, mb, gW, gb2, w1u, w1m, w1g, b12, W2, c2):
    return pl.pallas_call(
        _tc_body,
        grid=(B // BLK,),
        in_specs=[
            pl.BlockSpec((BLK, ED), lambda i: (i, 0)),
            pl.BlockSpec((BLK, ED), lambda i: (i, 0)),
            pl.BlockSpec((BLK, G), lambda i: (i, 0)),
            pl.BlockSpec((BLK,), lambda i: (i,)),
            pl.BlockSpec((BLK,), lambda i: (i,)),
            pl.BlockSpec((ED, G), lambda i: (0, 0)),
            pl.BlockSpec((1, ED), lambda i: (0, 0)),
            pl.BlockSpec((HL, ED), lambda i: (0, 0)),
            pl.BlockSpec((HL, ED), lambda i: (0, 0)),
            pl.BlockSpec((HL, ED), lambda i: (0, 0)),
            pl.BlockSpec((1, HL), lambda i: (0, 0)),
            pl.BlockSpec((1, HL), lambda i: (0, 0)),
            pl.BlockSpec((1, 1), lambda i: (0, 0)),
        ],
        out_specs=pl.BlockSpec((BLK,), lambda i: (i,)),
        out_shape=jax.ShapeDtypeStruct((B,), jnp.float32),
    )(urows, mrows, genre, ub, mb, gW, gb2, w1u, w1m, w1g, b12, W2, c2)


def kernel(userIndices, movieIndices, genreIndeces, userEmb, movieEmb,
           userBiasT, movieBiasT, bias, gW, gb, W1, b1, W2, b2):
    uidx = userIndices.astype(jnp.int32).reshape(NW, NCH, CHUNK)
    midx = movieIndices.astype(jnp.int32).reshape(NW, NCH, CHUNK)
    urows = jnp.take(userEmb, userIndices, axis=0)
    mrows, ub, mb = _sc_gather_fn()(
        uidx, midx, movieEmb, userBiasT[:, 0], movieBiasT[:, 0])
    w1u = W1[:, :ED]
    w1m = W1[:, ED:2 * ED]
    w1g = W1[:, 2 * ED:]
    c2 = (bias + b2).reshape(1, 1)
    return _tc_forward(urows, mrows, genreIndeces, ub, mb, gW,
                       gb.reshape(1, ED), w1u, w1m, w1g,
                       b1.reshape(1, HL), W2, c2)


# speedup vs baseline: 4.5216x; 1.1895x over previous
import functools

import jax
import jax.numpy as jnp
from jax import lax
from jax.experimental import pallas as pl
from jax.experimental.pallas import tpu as pltpu
from jax.experimental.pallas import tpu_sc as plsc

B = 16384
ED = 32
HL = 64
G = 20
NC, NS = 2, 16
NW = NC * NS
BPW = B // NW
CHUNK = 128
NCH = BPW // CHUNK
NG = BPW // 16


@functools.cache
def _sc_gather_fn():
    mesh = plsc.VectorSubcoreMesh(core_axis_name="c", subcore_axis_name="s",
                                  num_cores=NC, num_subcores=NS)

    @functools.partial(
        pl.kernel,
        out_type=(
            jax.ShapeDtypeStruct((ED, B // CHUNK, CHUNK), jnp.float32),
            jax.ShapeDtypeStruct((B,), jnp.float32),
            jax.ShapeDtypeStruct((B,), jnp.float32),
        ),
        mesh=mesh,
        compiler_params=pltpu.CompilerParams(use_tc_tiling_on_sc=False,
                                             needs_layout_passes=False),
        scratch_types=(
            pltpu.VMEM((NCH, CHUNK), jnp.int32),
            pltpu.VMEM((NCH, CHUNK), jnp.int32),
            pltpu.VMEM((NCH, CHUNK), jnp.int32),
            pltpu.VMEM((BPW, 4 * ED), jnp.float32),
            pltpu.VMEM((ED, NCH, CHUNK), jnp.float32),
            pltpu.VMEM((BPW,), jnp.float32),
            pltpu.VMEM((BPW,), jnp.float32),
            pltpu.SemaphoreType.DMA,
            pltpu.SemaphoreType.DMA,
        ),
    )
    def _sc_gather(uidx_hbm, midx_hbm, memb_hbm, ubias_hbm, mbias_hbm,
                   mrowsT_out, ub_out, mb_out,
                   uidx_v, midx_v, mphys_v, buf, mT_v, ub_v, mb_v, sem_g, sem_b):
        wid = lax.axis_index("s") * NC + lax.axis_index("c")
        base = wid * BPW
        pltpu.sync_copy(uidx_hbm.at[wid], uidx_v)
        pltpu.sync_copy(midx_hbm.at[wid], midx_v)
        bias_copies = []
        for j in range(NCH):
            sl = pl.ds(j * CHUNK, CHUNK)
            bias_copies.append(pltpu.async_copy(ubias_hbm.at[uidx_v.at[j]], ub_v.at[sl], sem_b))
            bias_copies.append(pltpu.async_copy(mbias_hbm.at[midx_v.at[j]], mb_v.at[sl], sem_b))
        for j in range(NCH):
            for k in range(CHUNK // 16):
                sl = pl.ds(k * 16, 16)
                mphys_v[j, sl] = lax.shift_right_logical(midx_v[j, sl], 2)
        g_copies = []
        for j in range(NCH):
            sl = pl.ds(j * CHUNK, CHUNK)
            g_copies.append(pltpu.async_copy(memb_hbm.at[mphys_v.at[j]], buf.at[sl], sem_g))
        for c in g_copies:
            c.wait()

        iota16 = lax.iota(jnp.int32, 16)

        def extract(g, carry):
            j = g >> 3
            gg = g & 7
            rowv = g * 16 + iota16
            selv = (midx_v[j, pl.ds(gg * 16, 16)] & 3) * ED
            for c in range(ED):
                v = plsc.load_gather(buf, [rowv, selv + c])
                mT_v[c, j, pl.ds(gg * 16, 16)] = v
            return carry

        lax.fori_loop(0, NG, extract, 0)

        for c in bias_copies:
            c.wait()
        pltpu.sync_copy(mT_v, mrowsT_out.at[:, pl.ds(wid * NCH, NCH), :])
        pltpu.sync_copy(ub_v, ub_out.at[pl.ds(base, BPW)])
        pltpu.sync_copy(mb_v, mb_out.at[pl.ds(base, BPW)])

    return _sc_gather


def _tc_body(uT_ref, mT_ref, gT_ref, ub_ref, mb_ref, gW_ref, gb_ref,
             w1u_ref, w1m_ref, w1g_ref, b1_ref, w2_ref, c2_ref, out_ref):
    cdims = (((1,), (0,)), ((), ()))
    uT = uT_ref[...]
    mT = mT_ref[...].reshape(ED, BLK)
    geT = lax.dot_general(gW_ref[...], gT_ref[...], cdims,
                          preferred_element_type=jnp.float32) + gb_ref[...]
    pre = lax.dot_general(w1u_ref[...], uT, cdims, preferred_element_type=jnp.float32)
    pre = pre + lax.dot_general(w1m_ref[...], mT, cdims, preferred_element_type=jnp.float32)
    pre = pre + lax.dot_general(w1g_ref[...], geT, cdims, preferred_element_type=jnp.float32)
    h = jnp.maximum(pre + b1_ref[...], 0.0)
    mlpT = lax.dot_general(w2_ref[...], h, cdims, preferred_element_type=jnp.float32)
    dp = jnp.sum(uT * mT, axis=0)
    out_ref[...] = dp + mlpT[0, :] + ub_ref[...] + mb_ref[...] + c2_ref[0, 0]


BLK = 2048


def _tc_forward(uT, mT, gT, ub, mb, gW, gb2, w1u, w1m, w1g, b12, W2, c2):
    return pl.pallas_call(
        _tc_body,
        grid=(B // BLK,),
        in_specs=[
            pl.BlockSpec((ED, BLK), lambda i: (0, i)),
            pl.BlockSpec((ED, BLK // CHUNK, CHUNK), lambda i: (0, i, 0)),
            pl.BlockSpec((G, BLK), lambda i: (0, i)),
            pl.BlockSpec((BLK,), lambda i: (i,)),
            pl.BlockSpec((BLK,), lambda i: (i,)),
            pl.BlockSpec((ED, G), lambda i: (0, 0)),
            pl.BlockSpec((ED, 1), lambda i: (0, 0)),
            pl.BlockSpec((HL, ED), lambda i: (0, 0)),
            pl.BlockSpec((HL, ED), lambda i: (0, 0)),
            pl.BlockSpec((HL, ED), lambda i: (0, 0)),
            pl.BlockSpec((HL, 1), lambda i: (0, 0)),
            pl.BlockSpec((1, HL), lambda i: (0, 0)),
            pl.BlockSpec((1, 1), lambda i: (0, 0)),
        ],
        out_specs=pl.BlockSpec((BLK,), lambda i: (i,)),
        out_shape=jax.ShapeDtypeStruct((B,), jnp.float32),
    )(uT, mT, gT, ub, mb, gW, gb2, w1u, w1m, w1g, b12, W2, c2)


def kernel(userIndices, movieIndices, genreIndeces, userEmb, movieEmb,
           userBiasT, movieBiasT, bias, gW, gb, W1, b1, W2, b2):
    uidx = userIndices.astype(jnp.int32).reshape(NW, NCH, CHUNK)
    midx = movieIndices.astype(jnp.int32).reshape(NW, NCH, CHUNK)
    urowsT = jnp.take(userEmb, userIndices, axis=0).T
    mT, ub, mb = _sc_gather_fn()(
        uidx, midx, movieEmb.reshape(-1, 4 * ED), userBiasT[:, 0], movieBiasT[:, 0])
    w1u = W1[:, :ED]
    w1m = W1[:, ED:2 * ED]
    w1g = W1[:, 2 * ED:]
    c2 = (bias + b2).reshape(1, 1)
    return _tc_forward(urowsT, mT, genreIndeces.T, ub, mb, gW,
                       gb.reshape(ED, 1), w1u, w1m, w1g,
                       b1.reshape(HL, 1), W2, c2)


# speedup vs baseline: 4.5401x; 1.0041x over previous
import functools

import jax
import jax.numpy as jnp
from jax import lax
from jax.experimental import pallas as pl
from jax.experimental.pallas import tpu as pltpu
from jax.experimental.pallas import tpu_sc as plsc

B = 16384
ED = 32
HL = 64
G = 20
NC, NS = 2, 16
NW = NC * NS
BPW = B // NW
CHUNK = 128
NCH = BPW // CHUNK
NG = BPW // 16


@functools.cache
def _sc_gather_fn():
    mesh = plsc.VectorSubcoreMesh(core_axis_name="c", subcore_axis_name="s",
                                  num_cores=NC, num_subcores=NS)

    @functools.partial(
        pl.kernel,
        out_type=(
            jax.ShapeDtypeStruct((ED, B // CHUNK, CHUNK), jnp.float32),
            jax.ShapeDtypeStruct((B,), jnp.float32),
            jax.ShapeDtypeStruct((B,), jnp.float32),
        ),
        mesh=mesh,
        compiler_params=pltpu.CompilerParams(use_tc_tiling_on_sc=False,
                                             needs_layout_passes=False),
        scratch_types=(
            pltpu.VMEM((NCH, CHUNK), jnp.int32),
            pltpu.VMEM((NCH, CHUNK), jnp.int32),
            pltpu.VMEM((NCH, CHUNK), jnp.int32),
            pltpu.VMEM((BPW, 4 * ED), jnp.float32),
            pltpu.VMEM((ED, NCH, CHUNK), jnp.float32),
            pltpu.VMEM((BPW,), jnp.float32),
            pltpu.VMEM((BPW,), jnp.float32),
            pltpu.SemaphoreType.DMA,
            pltpu.SemaphoreType.DMA,
        ),
    )
    def _sc_gather(uidx_hbm, midx_hbm, memb_hbm, ubias_hbm, mbias_hbm,
                   mrowsT_out, ub_out, mb_out,
                   uidx_v, midx_v, mphys_v, buf, mT_v, ub_v, mb_v, sem_g, sem_b):
        wid = lax.axis_index("s") * NC + lax.axis_index("c")
        base = wid * BPW
        pltpu.sync_copy(uidx_hbm.at[wid], uidx_v)
        pltpu.sync_copy(midx_hbm.at[wid], midx_v)
        bias_copies = []
        for j in range(NCH):
            sl = pl.ds(j * CHUNK, CHUNK)
            bias_copies.append(pltpu.async_copy(ubias_hbm.at[uidx_v.at[j]], ub_v.at[sl], sem_b))
            bias_copies.append(pltpu.async_copy(mbias_hbm.at[midx_v.at[j]], mb_v.at[sl], sem_b))
        for j in range(NCH):
            for k in range(CHUNK // 16):
                sl = pl.ds(k * 16, 16)
                mphys_v[j, sl] = lax.shift_right_logical(midx_v[j, sl], 2)
        g_copies = []
        for j in range(NCH):
            sl = pl.ds(j * CHUNK, CHUNK)
            g_copies.append(pltpu.async_copy(memb_hbm.at[mphys_v.at[j]], buf.at[sl], sem_g))
        for c in g_copies:
            c.wait()

        iota16 = lax.iota(jnp.int32, 16)

        def extract(g, carry):
            j = g >> 3
            gg = g & 7
            rowv = g * 16 + iota16
            selv = (midx_v[j, pl.ds(gg * 16, 16)] & 3) * ED
            for c in range(ED):
                v = plsc.load_gather(buf, [rowv, selv + c])
                mT_v[c, j, pl.ds(gg * 16, 16)] = v
            return carry

        lax.fori_loop(0, NG, extract, 0)

        for c in bias_copies:
            c.wait()
        pltpu.sync_copy(mT_v, mrowsT_out.at[:, pl.ds(wid * NCH, NCH), :])
        pltpu.sync_copy(ub_v, ub_out.at[pl.ds(base, BPW)])
        pltpu.sync_copy(mb_v, mb_out.at[pl.ds(base, BPW)])

    return _sc_gather


BLK = 2048


def _tc_body(uT_ref, mT_ref, gT_ref, ub_ref, mb_ref, gW_ref, gb_ref,
             w1_ref, b1_ref, w2_ref, c2_ref, out_ref):
    cdims = (((1,), (0,)), ((), ()))
    uT = uT_ref[...]
    mT = mT_ref[...].reshape(ED, BLK)
    w1 = w1_ref[...]
    geT = lax.dot_general(gW_ref[...], gT_ref[...], cdims,
                          preferred_element_type=jnp.float32) + gb_ref[...]
    pre = lax.dot_general(w1[:, :ED], uT, cdims, preferred_element_type=jnp.float32)
    pre = pre + lax.dot_general(w1[:, ED:2 * ED], mT, cdims,
                                preferred_element_type=jnp.float32)
    pre = pre + lax.dot_general(w1[:, 2 * ED:], geT, cdims,
                                preferred_element_type=jnp.float32)
    h = jnp.maximum(pre + b1_ref[...], 0.0)
    mlpT = lax.dot_general(w2_ref[...], h, cdims, preferred_element_type=jnp.float32)
    dp = jnp.sum(uT * mT, axis=0)
    out_ref[...] = dp + mlpT[0, :] + ub_ref[...] + mb_ref[...] + c2_ref[0, 0]


def _tc_forward(uT, mT, gT, ub, mb, gW, gb2, W1, b12, W2, c2):
    return pl.pallas_call(
        _tc_body,
        grid=(B // BLK,),
        in_specs=[
            pl.BlockSpec((ED, BLK), lambda i: (0, i)),
            pl.BlockSpec((ED, BLK // CHUNK, CHUNK), lambda i: (0, i, 0)),
            pl.BlockSpec((G, BLK), lambda i: (0, i)),
            pl.BlockSpec((BLK,), lambda i: (i,)),
            pl.BlockSpec((BLK,), lambda i: (i,)),
            pl.BlockSpec((ED, G), lambda i: (0, 0)),
            pl.BlockSpec((ED, 1), lambda i: (0, 0)),
            pl.BlockSpec((HL, 3 * ED), lambda i: (0, 0)),
            pl.BlockSpec((HL, 1), lambda i: (0, 0)),
            pl.BlockSpec((1, HL), lambda i: (0, 0)),
            pl.BlockSpec((1, 1), lambda i: (0, 0)),
        ],
        out_specs=pl.BlockSpec((BLK,), lambda i: (i,)),
        out_shape=jax.ShapeDtypeStruct((B,), jnp.float32),
    )(uT, mT, gT, ub, mb, gW, gb2, W1, b12, W2, c2)


def kernel(userIndices, movieIndices, genreIndeces, userEmb, movieEmb,
           userBiasT, movieBiasT, bias, gW, gb, W1, b1, W2, b2):
    uidx = userIndices.astype(jnp.int32).reshape(NW, NCH, CHUNK)
    midx = movieIndices.astype(jnp.int32).reshape(NW, NCH, CHUNK)
    urowsT = jnp.take(userEmb, userIndices, axis=0, mode="clip").T
    mT, ub, mb = _sc_gather_fn()(
        uidx, midx, movieEmb.reshape(-1, 4 * ED),
        userBiasT.reshape(-1), movieBiasT.reshape(-1))
    c2 = (bias + b2).reshape(1, 1)
    return _tc_forward(urowsT, mT, genreIndeces.T, ub, mb, gW,
                       gb.reshape(ED, 1), W1,
                       b1.reshape(HL, 1), W2, c2)


# speedup vs baseline: 4.6584x; 1.0261x over previous
import functools

import jax
import jax.numpy as jnp
from jax import lax
from jax.experimental import pallas as pl
from jax.experimental.pallas import tpu as pltpu
from jax.experimental.pallas import tpu_sc as plsc

B = 16384
ED = 32
HL = 64
G = 20
NC, NS = 2, 16
NW = NC * NS
BPW = B // NW
CHUNK = 128
NCH = BPW // CHUNK
NG = BPW // 16


@functools.cache
def _sc_gather_fn():
    mesh = plsc.VectorSubcoreMesh(core_axis_name="c", subcore_axis_name="s",
                                  num_cores=NC, num_subcores=NS)

    @functools.partial(
        pl.kernel,
        out_type=(
            jax.ShapeDtypeStruct((ED, B // CHUNK, CHUNK), jnp.float32),
            jax.ShapeDtypeStruct((B,), jnp.float32),
            jax.ShapeDtypeStruct((B,), jnp.float32),
        ),
        mesh=mesh,
        compiler_params=pltpu.CompilerParams(use_tc_tiling_on_sc=False,
                                             needs_layout_passes=False),
        scratch_types=(
            pltpu.VMEM((NCH, CHUNK), jnp.int32),
            pltpu.VMEM((NCH, CHUNK), jnp.int32),
            pltpu.VMEM((BPW, ED), jnp.float32),
            pltpu.VMEM((ED, NCH, CHUNK), jnp.float32),
            pltpu.VMEM((BPW,), jnp.float32),
            pltpu.VMEM((BPW,), jnp.float32),
            pltpu.SemaphoreType.DMA,
            pltpu.SemaphoreType.DMA,
        ),
    )
    def _sc_gather(uidx_hbm, midx_hbm, memb_hbm, ubias_hbm, mbias_hbm,
                   mrowsT_out, ub_out, mb_out,
                   uidx_v, midx_v, buf, mT_v, ub_v, mb_v, sem_g, sem_b):
        wid = lax.axis_index("s") * NC + lax.axis_index("c")
        base = wid * BPW
        pltpu.sync_copy(uidx_hbm.at[wid], uidx_v)
        pltpu.sync_copy(midx_hbm.at[wid], midx_v)
        bias_copies = []
        for j in range(NCH):
            sl = pl.ds(j * CHUNK, CHUNK)
            bias_copies.append(pltpu.async_copy(ubias_hbm.at[uidx_v.at[j]], ub_v.at[sl], sem_b))
            bias_copies.append(pltpu.async_copy(mbias_hbm.at[midx_v.at[j]], mb_v.at[sl], sem_b))
        g_copies = []
        for j in range(NCH):
            sl = pl.ds(j * CHUNK, CHUNK)
            g_copies.append(pltpu.async_copy(memb_hbm.at[midx_v.at[j]], buf.at[sl], sem_g))
        for c in g_copies:
            c.wait()

        iota16 = lax.iota(jnp.int32, 16)
        cvecs = [iota16 * 0 + c for c in range(ED)]

        def extract(g, carry):
            j = g >> 3
            gg = g & 7
            rowv = g * 16 + iota16
            for c in range(ED):
                v = plsc.load_gather(buf, [rowv, cvecs[c]])
                mT_v[c, j, pl.ds(gg * 16, 16)] = v
            return carry

        lax.fori_loop(0, NG, extract, 0, unroll=2)

        for c in bias_copies:
            c.wait()
        pltpu.sync_copy(mT_v, mrowsT_out.at[:, pl.ds(wid * NCH, NCH), :])
        pltpu.sync_copy(ub_v, ub_out.at[pl.ds(base, BPW)])
        pltpu.sync_copy(mb_v, mb_out.at[pl.ds(base, BPW)])

    return _sc_gather


BLK = 2048


def _tc_body(uT_ref, mT_ref, gT_ref, ub_ref, mb_ref, gW_ref, gb_ref,
             w1_ref, b1_ref, w2_ref, c2_ref, out_ref):
    cdims = (((1,), (0,)), ((), ()))
    uT = uT_ref[...]
    mT = mT_ref[...].reshape(ED, BLK)
    w1 = w1_ref[...]
    geT = lax.dot_general(gW_ref[...], gT_ref[...], cdims,
                          preferred_element_type=jnp.float32) + gb_ref[...]
    pre = lax.dot_general(w1[:, :ED], uT, cdims, preferred_element_type=jnp.float32)
    pre = pre + lax.dot_general(w1[:, ED:2 * ED], mT, cdims,
                                preferred_element_type=jnp.float32)
    pre = pre + lax.dot_general(w1[:, 2 * ED:], geT, cdims,
                                preferred_element_type=jnp.float32)
    h = jnp.maximum(pre + b1_ref[...], 0.0)
    mlpT = lax.dot_general(w2_ref[...], h, cdims, preferred_element_type=jnp.float32)
    dp = jnp.sum(uT * mT, axis=0)
    out_ref[...] = dp + mlpT[0, :] + ub_ref[...] + mb_ref[...] + c2_ref[0, 0]


def _tc_forward(uT, mT, gT, ub, mb, gW, gb2, W1, b12, W2, c2):
    return pl.pallas_call(
        _tc_body,
        grid=(B // BLK,),
        in_specs=[
            pl.BlockSpec((ED, BLK), lambda i: (0, i)),
            pl.BlockSpec((ED, BLK // CHUNK, CHUNK), lambda i: (0, i, 0)),
            pl.BlockSpec((G, BLK), lambda i: (0, i)),
            pl.BlockSpec((BLK,), lambda i: (i,)),
            pl.BlockSpec((BLK,), lambda i: (i,)),
            pl.BlockSpec((ED, G), lambda i: (0, 0)),
            pl.BlockSpec((ED, 1), lambda i: (0, 0)),
            pl.BlockSpec((HL, 3 * ED), lambda i: (0, 0)),
            pl.BlockSpec((HL, 1), lambda i: (0, 0)),
            pl.BlockSpec((1, HL), lambda i: (0, 0)),
            pl.BlockSpec((1, 1), lambda i: (0, 0)),
        ],
        out_specs=pl.BlockSpec((BLK,), lambda i: (i,)),
        out_shape=jax.ShapeDtypeStruct((B,), jnp.float32),
    )(uT, mT, gT, ub, mb, gW, gb2, W1, b12, W2, c2)


def kernel(userIndices, movieIndices, genreIndeces, userEmb, movieEmb,
           userBiasT, movieBiasT, bias, gW, gb, W1, b1, W2, b2):
    uidx = userIndices.astype(jnp.int32).reshape(NW, NCH, CHUNK)
    midx = movieIndices.astype(jnp.int32).reshape(NW, NCH, CHUNK)
    urowsT = jnp.take(userEmb, userIndices, axis=0, mode="clip").T
    mT, ub, mb = _sc_gather_fn()(
        uidx, midx, movieEmb,
        userBiasT.reshape(-1), movieBiasT.reshape(-1))
    c2 = (bias + b2).reshape(1, 1)
    return _tc_forward(urowsT, mT, genreIndeces.T, ub, mb, gW,
                       gb.reshape(ED, 1), W1,
                       b1.reshape(HL, 1), W2, c2)
